# Initial kernel scaffold; baseline (speedup 1.0000x reference)
import jax
import jax.numpy as jnp
from jax.experimental import pallas as pl


def kernel(x_v, x_f, edge_dual_v, edge_dual_f, W_v1, b_v1, W_v2, b_v2, W_f1, b_f1, W_f2, b_f2):
    raise NotImplementedError("write your pallas kernel here")


# speedup vs baseline: 2.1002x; 2.1002x over previous
import functools

import jax
import jax.numpy as jnp
from jax import lax
from jax.experimental import pallas as pl
from jax.experimental.pallas import tpu as pltpu
from jax.experimental.pallas import tpu_sc as plsc

_D = 128
_LANES = 16
_NSUB = 16
_CHUNK = 80
_CW = 16
_ZROWS = 40


def _round8(x):
  return ((x + 7) // 8) * 8


def _zero_2d(ref, rows, width):
  z = jnp.zeros((_LANES,), jnp.float32)

  def body(r, _):
    for c in range(width // _LANES):
      ref[r, pl.ds(c * _LANES, _LANES)] = z
    return ()

  lax.fori_loop(0, rows, body, ())


def _fill_iota(si, start):
  for k in range(_CHUNK // _LANES):
    si[pl.ds(k * _LANES, _LANES)] = (
        lax.iota(jnp.int32, _LANES) + (start + k * _LANES))


def _sc_side(tbl, gidx_hbm, sidx_hbm, sum_out, cnt_out, acc, cntacc,
             gb0, lbuf, ob, gi0, si0, si2, si3, si8, one8, sem0,
             sid, n_seg, seg_pad, e):
  per_tile_rows = seg_pad // _NSUB
  cnt_rows = seg_pad // 8 // _NSUB
  out_main = _round8((n_seg + _NSUB - 1) // _NSUB)
  out_last = n_seg - (_NSUB - 1) * out_main
  per_tile_e = e // _NSUB
  n_chunks = per_tile_e // _CHUNK

  _zero_2d(gb0, _CHUNK, _D)
  _zero_2d(ob, _LANES, _D)
  one_v = jnp.where(lax.iota(jnp.int32, _LANES) == 0, 1.0, 0.0)
  for j in range(8):
    ob[j, pl.ds(j * _LANES, _LANES)] = one_v

  zbase = sid * per_tile_rows

  def zero_body(j, _):
    _fill_iota(si0, zbase + j * _CHUNK)
    pltpu.sync_copy(gb0, acc.at[si0])
    return ()

  lax.fori_loop(0, per_tile_rows // _CHUNK, zero_body, ())

  _fill_iota(si0, sid * cnt_rows)
  pltpu.sync_copy(gb0, cntacc.at[si0])

  si8[pl.ds(0, _LANES)] = lax.iota(jnp.int32, _LANES)

  @pl.when(sid == 0)
  def _():
    pltpu.sync_copy(ob, one8.at[si8])

  plsc.subcore_barrier()

  ebase = sid * per_tile_e

  def chunk_body(j, _):
    off = ebase + j * _CHUNK
    pltpu.sync_copy(gidx_hbm.at[pl.ds(off, _CHUNK)], gi0)
    pltpu.sync_copy(sidx_hbm.at[pl.ds(off, _CHUNK)], si0)
    cp = pltpu.async_copy(tbl.at[gi0], gb0, sem0)
    for i in range(_CHUNK // _LANES):
      sg = si0[pl.ds(i * _LANES, _LANES)]
      si2[pl.ds(i * _LANES, _LANES)] = jnp.bitwise_and(sg, 7)
      si3[pl.ds(i * _LANES, _LANES)] = lax.shift_right_logical(sg, 3)
    pltpu.sync_copy(one8.at[si2], lbuf)
    cp.wait()
    pltpu.sync_copy(gb0, acc.at[si0], add=True)
    pltpu.sync_copy(lbuf, cntacc.at[si3], add=True)
    return ()

  lax.fori_loop(0, n_chunks, chunk_body, ())

  plsc.subcore_barrier()

  obase = sid * out_main

  def write_rows(total):
    done = 0
    while done < total:
      c = min(_CHUNK, total - done)
      _fill_iota(si0, obase + done)
      pltpu.sync_copy(acc.at[si0], gb0)
      pltpu.sync_copy(gb0.at[pl.ds(0, c)],
                      sum_out.at[pl.ds(obase + done, c)])
      done += c

  @pl.when(sid < _NSUB - 1)
  def _():
    write_rows(out_main)

  @pl.when(sid == _NSUB - 1)
  def _():
    write_rows(out_last)

  _fill_iota(si0, sid * cnt_rows)
  pltpu.sync_copy(cntacc.at[si0], gb0)
  pltpu.sync_copy(gb0, cnt_out.at[pl.ds(sid * cnt_rows, cnt_rows)])


def _make_sc_kernel(m, n, e):
  per_tile = -(-(max(m, n) + 1 + _NSUB - 1) // _NSUB // _CHUNK) * _CHUNK
  seg_pad = _NSUB * per_tile
  mesh = plsc.VectorSubcoreMesh(core_axis_name="c", subcore_axis_name="s")

  @functools.partial(
      pl.kernel,
      out_type=[
          jax.ShapeDtypeStruct((m, _D), jnp.float32),
          jax.ShapeDtypeStruct((seg_pad // 8, _D), jnp.float32),
          jax.ShapeDtypeStruct((n, _D), jnp.float32),
          jax.ShapeDtypeStruct((seg_pad // 8, _D), jnp.float32),
      ],
      mesh=mesh,
      scratch_types=[
          pltpu.VMEM((_CHUNK, _D), jnp.float32),
          pltpu.VMEM((_CHUNK, _D), jnp.float32),
          pltpu.VMEM((_LANES, _D), jnp.float32),
          pltpu.VMEM((_CHUNK,), jnp.int32),
          pltpu.VMEM((_CHUNK,), jnp.int32),
          pltpu.VMEM((_CHUNK,), jnp.int32),
          pltpu.VMEM((_CHUNK,), jnp.int32),
          pltpu.VMEM((_LANES,), jnp.int32),
          pltpu.VMEM_SHARED((_LANES, _D), jnp.float32),
          pltpu.VMEM_SHARED((seg_pad, _D), jnp.float32),
          pltpu.VMEM_SHARED((seg_pad // 8, _D), jnp.float32),
          pltpu.SemaphoreType.DMA,
      ],
  )
  def sc_kernel(x_v, x_f, col, row_t, row, col_t,
                sum_v, cnt_v, sum_f, cnt_f,
                gb0, lbuf, ob, gi0, si0, si2, si3, si8,
                one8, acc, cntacc, sem0):
    cid = lax.axis_index("c")
    sid = lax.axis_index("s")

    @pl.when(cid == 0)
    def _():
      _sc_side(x_f, col, row_t, sum_v, cnt_v, acc, cntacc,
               gb0, lbuf, ob, gi0, si0, si2, si3, si8, one8, sem0,
               sid, m, seg_pad, e)

    @pl.when(cid == 1)
    def _():
      _sc_side(x_v, row, col_t, sum_f, cnt_f, acc, cntacc,
               gb0, lbuf, ob, gi0, si0, si2, si3, si8, one8, sem0,
               sid, n, seg_pad, e)

  return sc_kernel


def _mlp_body(x_ref, s_ref, c_ref, w1a_ref, w1b_ref, b1_ref, w2_ref, b2_ref,
              o_ref):
  cnt = jnp.maximum(jnp.sum(c_ref[...], axis=1, keepdims=True), 1.0)
  agg = s_ref[...] / cnt
  h = (jnp.dot(x_ref[...], w1a_ref[...], preferred_element_type=jnp.float32)
       + jnp.dot(agg, w1b_ref[...], preferred_element_type=jnp.float32)
       + b1_ref[...])
  h = jnp.where(h >= 0, h, 0.2 * h)
  o = jnp.dot(h, w2_ref[...], preferred_element_type=jnp.float32) + b2_ref[...]
  o_ref[...] = jnp.where(o >= 0, o, 0.2 * o)


def _mlp(x, s, c, w1, b1, w2, b2):
  m = x.shape[0]
  bm = 1000
  grid = (m // bm,)
  w1a = w1[:_D]
  w1b = w1[_D:]
  return pl.pallas_call(
      _mlp_body,
      grid=grid,
      in_specs=[
          pl.BlockSpec((bm, _D), lambda i: (i, 0)),
          pl.BlockSpec((bm, _D), lambda i: (i, 0)),
          pl.BlockSpec((bm, _CW), lambda i: (i, 0)),
          pl.BlockSpec((_D, _D), lambda i: (0, 0)),
          pl.BlockSpec((_D, _D), lambda i: (0, 0)),
          pl.BlockSpec((1, _D), lambda i: (0, 0)),
          pl.BlockSpec((_D, _D), lambda i: (0, 0)),
          pl.BlockSpec((1, _D), lambda i: (0, 0)),
      ],
      out_specs=pl.BlockSpec((bm, _D), lambda i: (i, 0)),
      out_shape=jax.ShapeDtypeStruct((m, _D), jnp.float32),
  )(x, s, c, w1a, w1b, b1.reshape(1, _D), w2, b2.reshape(1, _D))


@jax.jit
def kernel(x_v, x_f, edge_dual_v, edge_dual_f,
           W_v1, b_v1, W_v2, b_v2, W_f1, b_f1, W_f2, b_f2):
  m = x_v.shape[0]
  n = x_f.shape[0]
  e = edge_dual_v.shape[0]

  lin = edge_dual_v.astype(jnp.int32) * n + edge_dual_f.astype(jnp.int32)
  lin = jnp.sort(lin)
  dup = jnp.concatenate([jnp.zeros((1,), bool), lin[1:] == lin[:-1]])
  row = lin // n
  col = lin % n
  row_t = jnp.where(dup, m, row).astype(jnp.int32)
  col_t = jnp.where(dup, n, col).astype(jnp.int32)
  row = row.astype(jnp.int32)
  col = col.astype(jnp.int32)

  sum_v, cnt_v, sum_f, cnt_f = _make_sc_kernel(m, n, e)(
      x_v, x_f, col, row_t, row, col_t)

  cnt_v = cnt_v.reshape(-1, _CW)[:m]
  cnt_f = cnt_f.reshape(-1, _CW)[:n]

  out_v = _mlp(x_v, sum_v, cnt_v, W_v1, b_v1, W_v2, b_v2)
  out_f = _mlp(x_f, sum_f, cnt_f, W_f1, b_f1, W_f2, b_f2)
  return (out_v, out_f)


# speedup vs baseline: 2.6155x; 1.2454x over previous
import functools

import jax
import jax.numpy as jnp
from jax import lax
from jax.experimental import pallas as pl
from jax.experimental.pallas import tpu as pltpu
from jax.experimental.pallas import tpu_sc as plsc

_D = 128
_LANES = 16
_NSUB = 16
_CHUNK = 80
_CW = 16
_ZROWS = 40


def _round8(x):
  return ((x + 7) // 8) * 8


def _zero_2d(ref, rows, width):
  z = jnp.zeros((_LANES,), jnp.float32)

  def body(r, _):
    for c in range(width // _LANES):
      ref[r, pl.ds(c * _LANES, _LANES)] = z
    return ()

  lax.fori_loop(0, rows, body, ())


def _fill_iota(si, start):
  for k in range(_CHUNK // _LANES):
    si[pl.ds(k * _LANES, _LANES)] = (
        lax.iota(jnp.int32, _LANES) + (start + k * _LANES))


def _sc_side(tbl, gidx_hbm, sidx_hbm, sum_out, cnt_out, acc, cntacc,
             gb0, gb1, lbuf, ob, gi0, gi1, si0, si1, si2, si3, si8, one8,
             sem0, sem1, sid, n_seg, seg_pad, e):
  per_tile_rows = seg_pad // _NSUB
  cnt_rows = seg_pad // 8 // _NSUB
  out_main = _round8((n_seg + _NSUB - 1) // _NSUB)
  out_last = n_seg - (_NSUB - 1) * out_main
  per_tile_e = e // _NSUB
  n_chunks = per_tile_e // _CHUNK

  _zero_2d(gb0, _CHUNK, _D)
  _zero_2d(ob, _LANES, _D)
  one_v = jnp.where(lax.iota(jnp.int32, _LANES) == 0, 1.0, 0.0)
  for j in range(8):
    ob[j, pl.ds(j * _LANES, _LANES)] = one_v

  zbase = sid * per_tile_rows

  def zero_body(j, _):
    _fill_iota(si0, zbase + j * _CHUNK)
    pltpu.sync_copy(gb0, acc.at[si0])
    return ()

  lax.fori_loop(0, per_tile_rows // _CHUNK, zero_body, ())

  _fill_iota(si0, sid * cnt_rows)
  pltpu.sync_copy(gb0, cntacc.at[si0])

  si8[pl.ds(0, _LANES)] = lax.iota(jnp.int32, _LANES)

  @pl.when(sid == 0)
  def _():
    pltpu.sync_copy(ob, one8.at[si8])

  plsc.subcore_barrier()

  ebase = sid * per_tile_e

  def count_rows(si):
    for i in range(_CHUNK // _LANES):
      sg = si[pl.ds(i * _LANES, _LANES)]
      si2[pl.ds(i * _LANES, _LANES)] = jnp.bitwise_and(sg, 7)
      si3[pl.ds(i * _LANES, _LANES)] = lax.shift_right_logical(sg, 3)
    pltpu.sync_copy(one8.at[si2], lbuf)

  def pair_body(k, _):
    off = ebase + 2 * k * _CHUNK
    pltpu.sync_copy(gidx_hbm.at[pl.ds(off, _CHUNK)], gi0)
    pltpu.sync_copy(sidx_hbm.at[pl.ds(off, _CHUNK)], si0)
    cp0 = pltpu.async_copy(tbl.at[gi0], gb0, sem0)
    pltpu.sync_copy(gidx_hbm.at[pl.ds(off + _CHUNK, _CHUNK)], gi1)
    pltpu.sync_copy(sidx_hbm.at[pl.ds(off + _CHUNK, _CHUNK)], si1)
    cp1 = pltpu.async_copy(tbl.at[gi1], gb1, sem1)
    count_rows(si0)
    cp0.wait()
    pltpu.sync_copy(gb0, acc.at[si0], add=True)
    pltpu.sync_copy(lbuf, cntacc.at[si3], add=True)
    count_rows(si1)
    cp1.wait()
    pltpu.sync_copy(gb1, acc.at[si1], add=True)
    pltpu.sync_copy(lbuf, cntacc.at[si3], add=True)
    return ()

  lax.fori_loop(0, n_chunks // 2, pair_body, ())

  plsc.subcore_barrier()

  obase = sid * out_main

  def write_rows(total):
    done = 0
    while done < total:
      c = min(_CHUNK, total - done)
      _fill_iota(si0, obase + done)
      pltpu.sync_copy(acc.at[si0], gb0)
      pltpu.sync_copy(gb0.at[pl.ds(0, c)],
                      sum_out.at[pl.ds(obase + done, c)])
      done += c

  @pl.when(sid < _NSUB - 1)
  def _():
    write_rows(out_main)

  @pl.when(sid == _NSUB - 1)
  def _():
    write_rows(out_last)

  _fill_iota(si0, sid * cnt_rows)
  pltpu.sync_copy(cntacc.at[si0], gb0)
  pltpu.sync_copy(gb0, cnt_out.at[pl.ds(sid * cnt_rows, cnt_rows)])


def _make_sc_kernel(m, n, e):
  per_tile = -(-(max(m, n) + 1 + _NSUB - 1) // _NSUB // _CHUNK) * _CHUNK
  seg_pad = _NSUB * per_tile
  mesh = plsc.VectorSubcoreMesh(core_axis_name="c", subcore_axis_name="s")

  @functools.partial(
      pl.kernel,
      out_type=[
          jax.ShapeDtypeStruct((m, _D), jnp.float32),
          jax.ShapeDtypeStruct((seg_pad // 8, _D), jnp.float32),
          jax.ShapeDtypeStruct((n, _D), jnp.float32),
          jax.ShapeDtypeStruct((seg_pad // 8, _D), jnp.float32),
      ],
      mesh=mesh,
      scratch_types=[
          pltpu.VMEM((_CHUNK, _D), jnp.float32),
          pltpu.VMEM((_CHUNK, _D), jnp.float32),
          pltpu.VMEM((_CHUNK, _D), jnp.float32),
          pltpu.VMEM((_LANES, _D), jnp.float32),
          pltpu.VMEM((_CHUNK,), jnp.int32),
          pltpu.VMEM((_CHUNK,), jnp.int32),
          pltpu.VMEM((_CHUNK,), jnp.int32),
          pltpu.VMEM((_CHUNK,), jnp.int32),
          pltpu.VMEM((_CHUNK,), jnp.int32),
          pltpu.VMEM((_CHUNK,), jnp.int32),
          pltpu.VMEM((_LANES,), jnp.int32),
          pltpu.VMEM_SHARED((_LANES, _D), jnp.float32),
          pltpu.VMEM_SHARED((seg_pad, _D), jnp.float32),
          pltpu.VMEM_SHARED((seg_pad // 8, _D), jnp.float32),
          pltpu.SemaphoreType.DMA,
          pltpu.SemaphoreType.DMA,
      ],
  )
  def sc_kernel(x_v, x_f, col, row_t, row, col_t,
                sum_v, cnt_v, sum_f, cnt_f,
                gb0, gb1, lbuf, ob, gi0, gi1, si0, si1, si2, si3, si8,
                one8, acc, cntacc, sem0, sem1):
    cid = lax.axis_index("c")
    sid = lax.axis_index("s")

    @pl.when(cid == 0)
    def _():
      _sc_side(x_f, col, row_t, sum_v, cnt_v, acc, cntacc,
               gb0, gb1, lbuf, ob, gi0, gi1, si0, si1, si2, si3, si8, one8,
               sem0, sem1, sid, m, seg_pad, e)

    @pl.when(cid == 1)
    def _():
      _sc_side(x_v, row, col_t, sum_f, cnt_f, acc, cntacc,
               gb0, gb1, lbuf, ob, gi0, gi1, si0, si1, si2, si3, si8, one8,
               sem0, sem1, sid, n, seg_pad, e)

  return sc_kernel


def _mlp_body(x_ref, s_ref, c_ref, w1a_ref, w1b_ref, b1_ref, w2_ref, b2_ref,
              o_ref):
  cnt = jnp.maximum(jnp.sum(c_ref[...], axis=1, keepdims=True), 1.0)
  agg = s_ref[...] / cnt
  h = (jnp.dot(x_ref[...], w1a_ref[...], preferred_element_type=jnp.float32)
       + jnp.dot(agg, w1b_ref[...], preferred_element_type=jnp.float32)
       + b1_ref[...])
  h = jnp.where(h >= 0, h, 0.2 * h)
  o = jnp.dot(h, w2_ref[...], preferred_element_type=jnp.float32) + b2_ref[...]
  o_ref[...] = jnp.where(o >= 0, o, 0.2 * o)


def _mlp(x, s, c, w1, b1, w2, b2):
  m = x.shape[0]
  bm = 1000
  grid = (m // bm,)
  w1a = w1[:_D]
  w1b = w1[_D:]
  return pl.pallas_call(
      _mlp_body,
      grid=grid,
      in_specs=[
          pl.BlockSpec((bm, _D), lambda i: (i, 0)),
          pl.BlockSpec((bm, _D), lambda i: (i, 0)),
          pl.BlockSpec((bm, _CW), lambda i: (i, 0)),
          pl.BlockSpec((_D, _D), lambda i: (0, 0)),
          pl.BlockSpec((_D, _D), lambda i: (0, 0)),
          pl.BlockSpec((1, _D), lambda i: (0, 0)),
          pl.BlockSpec((_D, _D), lambda i: (0, 0)),
          pl.BlockSpec((1, _D), lambda i: (0, 0)),
      ],
      out_specs=pl.BlockSpec((bm, _D), lambda i: (i, 0)),
      out_shape=jax.ShapeDtypeStruct((m, _D), jnp.float32),
  )(x, s, c, w1a, w1b, b1.reshape(1, _D), w2, b2.reshape(1, _D))


@jax.jit
def kernel(x_v, x_f, edge_dual_v, edge_dual_f,
           W_v1, b_v1, W_v2, b_v2, W_f1, b_f1, W_f2, b_f2):
  m = x_v.shape[0]
  n = x_f.shape[0]
  e = edge_dual_v.shape[0]

  lin = edge_dual_v.astype(jnp.int32) * n + edge_dual_f.astype(jnp.int32)
  lin = jnp.sort(lin)
  dup = jnp.concatenate([jnp.zeros((1,), bool), lin[1:] == lin[:-1]])
  row = lin // n
  col = lin % n
  row_t = jnp.where(dup, m, row).astype(jnp.int32)
  col_t = jnp.where(dup, n, col).astype(jnp.int32)
  row = row.astype(jnp.int32)
  col = col.astype(jnp.int32)

  sum_v, cnt_v, sum_f, cnt_f = _make_sc_kernel(m, n, e)(
      x_v, x_f, col, row_t, row, col_t)

  cnt_v = cnt_v.reshape(-1, _CW)[:m]
  cnt_f = cnt_f.reshape(-1, _CW)[:n]

  out_v = _mlp(x_v, sum_v, cnt_v, W_v1, b_v1, W_v2, b_v2)
  out_f = _mlp(x_f, sum_f, cnt_f, W_f1, b_f1, W_f2, b_f2)
  return (out_v, out_f)
